# baseline (speedup 1.0000x reference)
import functools
import math

import jax
import jax.numpy as jnp
from jax import lax
from jax.experimental import pallas as pl
from jax.experimental.pallas import tpu as pltpu
from jax.experimental.pallas import tpu_sc as plsc

H = 1024
B, T = 4, 8192
SCALE = math.sqrt(H)
NCOMBO = 729
NC, NS = 2, 16
NW = NC * NS
ROWS_PER_W = (B * T) // NW
CH = 32
NCH = ROWS_PER_W // CH
POW3 = (1, 3, 9, 27, 81, 243)


def _ctable_body(em_sm, sm_sm, pc_sm, rg_sm, wstack_r,
                 em_w, sm_w, pc_w, rg_w, out_r):
    b = pl.program_id(0)
    r = lax.broadcasted_iota(jnp.int32, (NCOMBO, 18), 0)
    cols = lax.broadcasted_iota(jnp.int32, (NCOMBO, 18), 1)
    k = cols // 3
    d = cols % 3
    pow3 = jnp.full_like(cols, POW3[5])
    for kk in range(4, -1, -1):
        pow3 = jnp.where(k == kk, POW3[kk], pow3)
    onehot = ((r // pow3) % 3 == d).astype(jnp.float32)
    x = jnp.dot(onehot, wstack_r[...], preferred_element_type=jnp.float32)
    bias = em_w[pl.ds(em_sm[b], 1), :]
    bias = bias + sm_w[pl.ds(sm_sm[b], 1), :]
    bias = bias + pc_w[pl.ds(pc_sm[b], 1), :]
    bias = bias + rg_w[pl.ds(rg_sm[b], 1), :]
    out_r[0] = (x + bias) * SCALE


def _build_ctable(emotion, singing_method, pace, range_, wstack,
                  emotion_W, singing_method_W, pace_W, range_W):
    smem = pl.BlockSpec(memory_space=pltpu.SMEM)
    full = lambda s: pl.BlockSpec(s, lambda b: (0,) * len(s))
    ct = pl.pallas_call(
        _ctable_body,
        grid=(B,),
        in_specs=[smem, smem, smem, smem,
                  full((18, H)), full((4, H)), full((4, H)), full((5, H)),
                  full((5, H))],
        out_specs=pl.BlockSpec((1, NCOMBO, H), lambda b: (b, 0, 0)),
        out_shape=jax.ShapeDtypeStruct((B, NCOMBO, H), jnp.float32),
    )(emotion, singing_method, pace, range_, wstack,
      emotion_W, singing_method_W, pace_W, range_W)
    return ct.reshape(B * NCOMBO, H)


def _sc_body(mix_h, fal_h, bre_h, pha_h, gli_h, vib_h, ct_h, out_h,
             idx6, cidx, rows, gs0, gs1, os0, os1):
    wid = lax.axis_index("s") * NC + lax.axis_index("c")
    base = pl.multiple_of(wid * ROWS_PER_W, ROWS_PER_W)
    b = wid // (NW // B)

    for t, h in enumerate((mix_h, fal_h, bre_h, pha_h, gli_h, vib_h)):
        pltpu.sync_copy(h.at[pl.ds(base, ROWS_PER_W)], idx6.at[t])
    for j in range(ROWS_PER_W // 16):
        s = pl.ds(j * 16, 16)
        v = idx6[0, s]
        for t in range(1, 6):
            v = v + idx6[t, s] * POW3[t]
        cidx[s] = v + b * NCOMBO

    gsems = (gs0, gs1)
    osems = (os0, os1)

    def _gather(i, buf):
        off = pl.multiple_of(i * CH, CH)
        return pltpu.make_async_copy(
            ct_h.at[cidx.at[pl.ds(off, CH)]], rows.at[buf], gsems[buf])

    def _out(i, buf):
        off = pl.multiple_of(base + i * CH, CH)
        return pltpu.make_async_copy(
            rows.at[buf], out_h.at[pl.ds(off, CH)], osems[buf])

    _gather(0, 0).start()
    _gather(1, 1).start()

    @pl.loop(0, NCH // 2 - 1)
    def _steady(s_):
        for j in range(2):
            i = s_ * 2 + j
            _gather(i, j).wait()
            _out(i, j).start()
            _out(i, j).wait()
            _gather(i + 2, j).start()

    for j in range(2):
        i = NCH - 2 + j
        _gather(i, j).wait()
        _out(i, j).start()
        _out(i, j).wait()


_sc_gather = functools.partial(
    pl.kernel,
    out_type=jax.ShapeDtypeStruct((B * T, H), jnp.float32),
    mesh=plsc.VectorSubcoreMesh(core_axis_name="c", subcore_axis_name="s",
                                num_cores=NC, num_subcores=NS),
    scratch_types=[
        pltpu.VMEM((6, ROWS_PER_W), jnp.int32),
        pltpu.VMEM((ROWS_PER_W,), jnp.int32),
        pltpu.VMEM((2, CH, H), jnp.float32),
        pltpu.SemaphoreType.DMA,
        pltpu.SemaphoreType.DMA,
        pltpu.SemaphoreType.DMA,
        pltpu.SemaphoreType.DMA,
    ],
)(_sc_body)


def kernel(mix, falsetto, breathy, pharyngeal, glissando, vibrato,
           emotion, singing_method, pace, range_,
           mix_W, falsetto_W, breathy_W, pharyngeal_W, glissando_W, vibrato_W,
           emotion_W, singing_method_W, pace_W, range_W):
    wstack = jnp.concatenate(
        [mix_W, falsetto_W, breathy_W, pharyngeal_W, glissando_W, vibrato_W],
        axis=0)
    ctable = _build_ctable(emotion, singing_method, pace, range_, wstack,
                           emotion_W, singing_method_W, pace_W, range_W)
    flat = [a.reshape(B * T) for a in
            (mix, falsetto, breathy, pharyngeal, glissando, vibrato)]
    out = _sc_gather(*flat, ctable)
    return out.reshape(B, T, H)
